# Initial kernel scaffold; baseline (speedup 1.0000x reference)
import jax
import jax.numpy as jnp
from jax.experimental import pallas as pl


def kernel(x, edge_index, batch, W1, b1, W2, b2):
    raise NotImplementedError("write your pallas kernel here")


# speedup vs baseline: 16.6912x; 16.6912x over previous
import functools

import jax
import jax.numpy as jnp
from jax import lax
from jax.experimental import pallas as pl
from jax.experimental.pallas import tpu as pltpu
from jax.experimental.pallas import tpu_sc as plsc

N = 10000
E = 320000
D = 128
H = 16
C = 10
G = 64

NC = 2
NS = 16
NW = NC * NS
CHUNK = 128
CPW = -(-E // (NW * CHUNK))
EPW = CPW * CHUNK
E_PAD = NW * EPW
N_PAD = 10240
RPW = N_PAD // NS
NB = N_PAD // 128


def _sc_deg_body(dst_hbm, out_hbm, zbuf, ones_v, idx_v, acc):
    c = lax.axis_index("c")
    s = lax.axis_index("s")
    wid = c * NS + s
    zero16 = jnp.zeros((16,), jnp.float32)
    one16 = jnp.ones((16,), jnp.float32)
    for j in range(CHUNK // 16):
        zbuf[pl.ds(j * 16, 16)] = zero16
        ones_v[pl.ds(j * 16, 16)] = one16
    base_r = s * RPW
    for j in range(RPW // CHUNK):
        pltpu.sync_copy(zbuf, acc.at[pl.ds(base_r + j * CHUNK, CHUNK)])
    plsc.subcore_barrier()
    ebase = wid * EPW
    for k in range(CPW):
        pltpu.sync_copy(dst_hbm.at[pl.ds(ebase + k * CHUNK, CHUNK)], idx_v)
        pltpu.sync_copy(ones_v, acc.at[idx_v], add=True)
    plsc.subcore_barrier()
    pltpu.sync_copy(acc.at[pl.ds(base_r, RPW)],
                    out_hbm.at[pl.ds(c * N_PAD + base_r, RPW)])


def _make_sc_deg():
    return pl.kernel(
        _sc_deg_body,
        out_type=jax.ShapeDtypeStruct((NC * N_PAD,), jnp.float32),
        mesh=plsc.VectorSubcoreMesh(core_axis_name="c", subcore_axis_name="s"),
        scratch_types=[
            pltpu.VMEM((CHUNK,), jnp.float32),
            pltpu.VMEM((CHUNK,), jnp.float32),
            pltpu.VMEM((CHUNK,), jnp.int32),
            pltpu.VMEM_SHARED((N_PAD,), jnp.float32),
        ],
        compiler_params=pltpu.CompilerParams(use_tc_tiling_on_sc=False),
        name="sc_degree",
    )


def _sc_agg_body(feat_hbm, src_hbm, dst_hbm, out_hbm,
                 zbuf, src_v, dst_v, rows_v, sem, acc):
    c = lax.axis_index("c")
    s = lax.axis_index("s")
    wid = c * NS + s
    zero16 = jnp.zeros((16,), jnp.float32)
    for j in range(CHUNK):
        zbuf[j, :] = zero16
    base_r = s * RPW
    for j in range(RPW // CHUNK):
        pltpu.sync_copy(zbuf, acc.at[pl.ds(base_r + j * CHUNK, CHUNK)])
    plsc.subcore_barrier()
    ebase = wid * EPW
    for k in range(CPW):
        off = ebase + k * CHUNK
        pltpu.sync_copy(src_hbm.at[pl.ds(off, CHUNK)], src_v)
        pltpu.sync_copy(dst_hbm.at[pl.ds(off, CHUNK)], dst_v)
        pltpu.async_copy(feat_hbm.at[src_v], rows_v, sem).wait()
        pltpu.sync_copy(rows_v, acc.at[dst_v], add=True)
    plsc.subcore_barrier()
    pltpu.sync_copy(acc.at[pl.ds(base_r, RPW)],
                    out_hbm.at[pl.ds(c * N_PAD + base_r, RPW)])


def _make_sc_agg():
    return pl.kernel(
        _sc_agg_body,
        out_type=jax.ShapeDtypeStruct((NC * N_PAD, H), jnp.float32),
        mesh=plsc.VectorSubcoreMesh(core_axis_name="c", subcore_axis_name="s"),
        scratch_types=[
            pltpu.VMEM((CHUNK, H), jnp.float32),
            pltpu.VMEM((CHUNK,), jnp.int32),
            pltpu.VMEM((CHUNK,), jnp.int32),
            pltpu.VMEM((CHUNK, H), jnp.float32),
            pltpu.SemaphoreType.DMA,
            pltpu.VMEM_SHARED((N_PAD, H), jnp.float32),
        ],
        compiler_params=pltpu.CompilerParams(use_tc_tiling_on_sc=False),
        name="sc_edge_agg",
    )


def _tc1_body(x_ref, w1_ref, d0_ref, d1_ref, dinv_ref, h1s_ref):
    i = pl.program_id(0)
    deg = d0_ref[...] + d1_ref[...] + 1.0
    rows = lax.broadcasted_iota(jnp.int32, (128, 1), 0) + i * 128
    dinv = jnp.where(rows < N, lax.rsqrt(deg), 0.0)
    h1 = jnp.dot(x_ref[...], w1_ref[...], preferred_element_type=jnp.float32, precision=lax.Precision.HIGHEST)
    dinv_ref[...] = dinv
    h1s_ref[...] = dinv * h1


def _tc1(x_pad, w1, d0, d1):
    return pl.pallas_call(
        _tc1_body,
        grid=(NB,),
        in_specs=[
            pl.BlockSpec((128, D), lambda i: (i, 0)),
            pl.BlockSpec((D, H), lambda i: (0, 0)),
            pl.BlockSpec((128, 1), lambda i: (i, 0)),
            pl.BlockSpec((128, 1), lambda i: (i, 0)),
        ],
        out_specs=[
            pl.BlockSpec((128, 1), lambda i: (i, 0)),
            pl.BlockSpec((128, H), lambda i: (i, 0)),
        ],
        out_shape=[
            jax.ShapeDtypeStruct((N_PAD, 1), jnp.float32),
            jax.ShapeDtypeStruct((N_PAD, H), jnp.float32),
        ],
        name="tc_dinv_h1s",
    )(x_pad, w1, d0, d1)


def _tc2_body(z1a_ref, z1b_ref, h1s_ref, dinv_ref, b1_ref, w2_ref, ys_ref):
    dinv = dinv_ref[...]
    out1 = dinv * (z1a_ref[...] + z1b_ref[...] + h1s_ref[...]) + b1_ref[...]
    h = jnp.maximum(out1, 0.0)
    ys_ref[...] = dinv * jnp.dot(h, w2_ref[...],
                                 preferred_element_type=jnp.float32, precision=lax.Precision.HIGHEST)


def _tc2(z1a, z1b, h1s, dinv, b1, w2p):
    return pl.pallas_call(
        _tc2_body,
        grid=(NB,),
        in_specs=[
            pl.BlockSpec((128, H), lambda i: (i, 0)),
            pl.BlockSpec((128, H), lambda i: (i, 0)),
            pl.BlockSpec((128, H), lambda i: (i, 0)),
            pl.BlockSpec((128, 1), lambda i: (i, 0)),
            pl.BlockSpec((1, H), lambda i: (0, 0)),
            pl.BlockSpec((H, H), lambda i: (0, 0)),
        ],
        out_specs=pl.BlockSpec((128, H), lambda i: (i, 0)),
        out_shape=jax.ShapeDtypeStruct((N_PAD, H), jnp.float32),
        name="tc_layer1_finish",
    )(z1a, z1b, h1s, dinv, b1, w2p)


def _tc3_body(z2a_ref, z2b_ref, ys_ref, dinv_ref, b2_ref, batch_ref,
              out_ref, pool_acc, cnt_acc):
    i = pl.program_id(0)
    dinv = dinv_ref[...]
    out2 = dinv * (z2a_ref[...] + z2b_ref[...] + ys_ref[...]) + b2_ref[...]
    bt = batch_ref[...].reshape(1, 128)
    gid = lax.broadcasted_iota(jnp.int32, (G, 1), 0)
    oh = (bt == gid).astype(jnp.float32)
    pool_i = jnp.dot(oh, out2, preferred_element_type=jnp.float32, precision=lax.Precision.HIGHEST)
    cnt_i = jnp.sum(oh, axis=1, keepdims=True)

    @pl.when(i == 0)
    def _():
        pool_acc[...] = pool_i
        cnt_acc[...] = cnt_i

    @pl.when(i > 0)
    def _():
        pool_acc[...] = pool_acc[...] + pool_i
        cnt_acc[...] = cnt_acc[...] + cnt_i

    @pl.when(i == NB - 1)
    def _():
        pooled = pool_acc[...] / jnp.maximum(cnt_acc[...], 1.0)
        col = lax.broadcasted_iota(jnp.int32, (G, H), 1)
        valid = col < C
        mx = jnp.max(jnp.where(valid, pooled, -1e30), axis=1, keepdims=True)
        ex = jnp.where(valid, jnp.exp(pooled - mx), 0.0)
        lse = mx + jnp.log(jnp.sum(ex, axis=1, keepdims=True))
        out_ref[...] = (pooled - lse)[:, :C]


def _tc3(z2a, z2b, ys, dinv, b2p, batch_r):
    return pl.pallas_call(
        _tc3_body,
        grid=(NB,),
        in_specs=[
            pl.BlockSpec((128, H), lambda i: (i, 0)),
            pl.BlockSpec((128, H), lambda i: (i, 0)),
            pl.BlockSpec((128, H), lambda i: (i, 0)),
            pl.BlockSpec((128, 1), lambda i: (i, 0)),
            pl.BlockSpec((1, H), lambda i: (0, 0)),
            pl.BlockSpec((1, 1, 128), lambda i: (i, 0, 0)),
        ],
        out_specs=pl.BlockSpec((G, C), lambda i: (0, 0)),
        out_shape=jax.ShapeDtypeStruct((G, C), jnp.float32),
        scratch_shapes=[
            pltpu.VMEM((G, H), jnp.float32),
            pltpu.VMEM((G, 1), jnp.float32),
        ],
        name="tc_pool_logsoftmax",
    )(z2a, z2b, ys, dinv, b2p, batch_r)


def kernel(x, edge_index, batch, W1, b1, W2, b2):
    f32 = jnp.float32
    src = edge_index[0].astype(jnp.int32)
    dst = edge_index[1].astype(jnp.int32)
    npad_e = E_PAD - E
    pad_rows = N + (jnp.arange(npad_e, dtype=jnp.int32) % (N_PAD - N))
    src_pad = jnp.concatenate([src, pad_rows])
    dst_pad = jnp.concatenate([dst, pad_rows])
    x_pad = jnp.zeros((N_PAD, D), f32).at[:N].set(x.astype(f32))
    batch_pad = jnp.full((N_PAD,), G, jnp.int32).at[:N].set(batch.astype(jnp.int32))
    batch_r = batch_pad.reshape(NB, 1, 128)
    w2p = jnp.zeros((H, H), f32).at[:, :C].set(W2.astype(f32))
    b1r = b1.astype(f32).reshape(1, H)
    b2p = jnp.zeros((1, H), f32).at[0, :C].set(b2.astype(f32))

    deg_parts = _make_sc_deg()(dst_pad)
    d0 = deg_parts[:N_PAD].reshape(N_PAD, 1)
    d1 = deg_parts[N_PAD:].reshape(N_PAD, 1)

    dinv, h1s = _tc1(x_pad, W1.astype(f32), d0, d1)

    sc_agg = _make_sc_agg()
    z1 = sc_agg(h1s, src_pad, dst_pad)
    z1a, z1b = z1[:N_PAD], z1[N_PAD:]

    ys = _tc2(z1a, z1b, h1s, dinv, b1r, w2p)

    z2 = sc_agg(ys, src_pad, dst_pad)
    z2a, z2b = z2[:N_PAD], z2[N_PAD:]

    return _tc3(z2a, z2b, ys, dinv, b2p, batch_r)


# speedup vs baseline: 33.6750x; 2.0175x over previous
import functools

import jax
import jax.numpy as jnp
from jax import lax
from jax.experimental import pallas as pl
from jax.experimental.pallas import tpu as pltpu
from jax.experimental.pallas import tpu_sc as plsc

N = 10000
E = 320000
D = 128
H = 16
C = 10
G = 64

NC = 2
NS = 16
NW = NC * NS
CHUNK = 128
CPW = -(-E // (NW * CHUNK))
EPW = CPW * CHUNK
E_PAD = NW * EPW
N_PAD = 10240
RPW = N_PAD // NS
NB = N_PAD // 128


NBUF = 8
LEAD = 4


def _sc_deg_body(idx_hbm, out_hbm, zbuf, ones_v, dslab, acc, *sems):
    c = lax.axis_index("c")
    s = lax.axis_index("s")
    wid = c * NS + s
    zero16 = jnp.zeros((16,), jnp.float32)
    one16 = jnp.ones((16,), jnp.float32)
    for j in range(CHUNK // 16):
        zbuf[pl.ds(j * 16, 16)] = zero16
        ones_v[pl.ds(j * 16, 16)] = one16
    pltpu.sync_copy(idx_hbm.at[pl.ds((NW + wid) * CPW, CPW)], dslab)
    base_r = s * RPW
    for j in range(RPW // CHUNK):
        pltpu.sync_copy(zbuf, acc.at[pl.ds(base_r + j * CHUNK, CHUNK)])
    plsc.subcore_barrier()
    sd = {}
    for k in range(CPW):
        if k >= NBUF:
            sd[k - NBUF].wait()
        sd[k] = pltpu.async_copy(ones_v, acc.at[dslab.at[k]], sems[k % NBUF],
                                 add=True)
    for k in range(max(0, CPW - NBUF), CPW):
        sd[k].wait()
    plsc.subcore_barrier()
    pltpu.sync_copy(acc.at[pl.ds(base_r, RPW)],
                    out_hbm.at[pl.ds(c * N_PAD + base_r, RPW)])


def _make_sc_deg():
    return pl.kernel(
        _sc_deg_body,
        out_type=jax.ShapeDtypeStruct((NC * N_PAD,), jnp.float32),
        mesh=plsc.VectorSubcoreMesh(core_axis_name="c", subcore_axis_name="s"),
        scratch_types=[
            pltpu.VMEM((CHUNK,), jnp.float32),
            pltpu.VMEM((CHUNK,), jnp.float32),
            pltpu.VMEM((CPW, CHUNK), jnp.int32),
            pltpu.VMEM_SHARED((N_PAD,), jnp.float32),
        ] + [pltpu.SemaphoreType.DMA] * NBUF,
        compiler_params=pltpu.CompilerParams(use_tc_tiling_on_sc=False),
        name="sc_degree",
    )


def _sc_agg_body(feat_hbm, idx_hbm, out_hbm,
                 zbuf, sslab, dslab, rows, acc, *sems):
    gsem = sems[:NBUF]
    ssem = sems[NBUF:]
    c = lax.axis_index("c")
    s = lax.axis_index("s")
    wid = c * NS + s
    zero16 = jnp.zeros((16,), jnp.float32)
    for j in range(CHUNK):
        zbuf[j, :] = zero16
    pltpu.sync_copy(idx_hbm.at[pl.ds(wid * CPW, CPW)], sslab)
    pltpu.sync_copy(idx_hbm.at[pl.ds((NW + wid) * CPW, CPW)], dslab)
    base_r = s * RPW
    for j in range(RPW // CHUNK):
        pltpu.sync_copy(zbuf, acc.at[pl.ds(base_r + j * CHUNK, CHUNK)])
    plsc.subcore_barrier()

    gd, sd = {}, {}
    s_waited = set()

    def start_gather(k):
        b = k % NBUF
        gd[k] = pltpu.async_copy(feat_hbm.at[sslab.at[k]], rows.at[b], gsem[b])

    for j in range(min(LEAD, CPW)):
        start_gather(j)
    for k in range(CPW):
        b = k % NBUF
        nk = k + LEAD
        if nk < CPW:
            if nk >= NBUF:
                sd[nk - NBUF].wait()
                s_waited.add(nk - NBUF)
            start_gather(nk)
        gd[k].wait()
        sd[k] = pltpu.async_copy(rows.at[b], acc.at[dslab.at[k]], ssem[b],
                                 add=True)
    for k in range(CPW):
        if k not in s_waited:
            sd[k].wait()
    plsc.subcore_barrier()
    pltpu.sync_copy(acc.at[pl.ds(base_r, RPW)],
                    out_hbm.at[pl.ds(c * N_PAD + base_r, RPW)])


def _make_sc_agg():
    return pl.kernel(
        _sc_agg_body,
        out_type=jax.ShapeDtypeStruct((NC * N_PAD, H), jnp.float32),
        mesh=plsc.VectorSubcoreMesh(core_axis_name="c", subcore_axis_name="s"),
        scratch_types=[
            pltpu.VMEM((CHUNK, H), jnp.float32),
            pltpu.VMEM((CPW, CHUNK), jnp.int32),
            pltpu.VMEM((CPW, CHUNK), jnp.int32),
            pltpu.VMEM((NBUF, CHUNK, H), jnp.float32),
            pltpu.VMEM_SHARED((N_PAD, H), jnp.float32),
        ] + [pltpu.SemaphoreType.DMA] * (2 * NBUF),
        compiler_params=pltpu.CompilerParams(use_tc_tiling_on_sc=False),
        name="sc_edge_agg",
    )


def _tc1_body(x_ref, w1_ref, d0_ref, d1_ref, dinv_ref, h1s_ref):
    i = pl.program_id(0)
    deg = d0_ref[...] + d1_ref[...] + 1.0
    rows = lax.broadcasted_iota(jnp.int32, (128, 1), 0) + i * 128
    dinv = jnp.where(rows < N, lax.rsqrt(deg), 0.0)
    h1 = jnp.dot(x_ref[...], w1_ref[...], preferred_element_type=jnp.float32, precision=lax.Precision.HIGHEST)
    dinv_ref[...] = dinv
    h1s_ref[...] = dinv * h1


def _tc1(x_pad, w1, d0, d1):
    return pl.pallas_call(
        _tc1_body,
        grid=(NB,),
        in_specs=[
            pl.BlockSpec((128, D), lambda i: (i, 0)),
            pl.BlockSpec((D, H), lambda i: (0, 0)),
            pl.BlockSpec((128, 1), lambda i: (i, 0)),
            pl.BlockSpec((128, 1), lambda i: (i, 0)),
        ],
        out_specs=[
            pl.BlockSpec((128, 1), lambda i: (i, 0)),
            pl.BlockSpec((128, H), lambda i: (i, 0)),
        ],
        out_shape=[
            jax.ShapeDtypeStruct((N_PAD, 1), jnp.float32),
            jax.ShapeDtypeStruct((N_PAD, H), jnp.float32),
        ],
        name="tc_dinv_h1s",
    )(x_pad, w1, d0, d1)


def _tc2_body(z1a_ref, z1b_ref, h1s_ref, dinv_ref, b1_ref, w2_ref, ys_ref):
    dinv = dinv_ref[...]
    out1 = dinv * (z1a_ref[...] + z1b_ref[...] + h1s_ref[...]) + b1_ref[...]
    h = jnp.maximum(out1, 0.0)
    ys_ref[...] = dinv * jnp.dot(h, w2_ref[...],
                                 preferred_element_type=jnp.float32, precision=lax.Precision.HIGHEST)


def _tc2(z1a, z1b, h1s, dinv, b1, w2p):
    return pl.pallas_call(
        _tc2_body,
        grid=(NB,),
        in_specs=[
            pl.BlockSpec((128, H), lambda i: (i, 0)),
            pl.BlockSpec((128, H), lambda i: (i, 0)),
            pl.BlockSpec((128, H), lambda i: (i, 0)),
            pl.BlockSpec((128, 1), lambda i: (i, 0)),
            pl.BlockSpec((1, H), lambda i: (0, 0)),
            pl.BlockSpec((H, H), lambda i: (0, 0)),
        ],
        out_specs=pl.BlockSpec((128, H), lambda i: (i, 0)),
        out_shape=jax.ShapeDtypeStruct((N_PAD, H), jnp.float32),
        name="tc_layer1_finish",
    )(z1a, z1b, h1s, dinv, b1, w2p)


def _tc3_body(z2a_ref, z2b_ref, ys_ref, dinv_ref, b2_ref, batch_ref,
              out_ref, pool_acc, cnt_acc):
    i = pl.program_id(0)
    dinv = dinv_ref[...]
    out2 = dinv * (z2a_ref[...] + z2b_ref[...] + ys_ref[...]) + b2_ref[...]
    bt = batch_ref[...].reshape(1, 128)
    gid = lax.broadcasted_iota(jnp.int32, (G, 1), 0)
    oh = (bt == gid).astype(jnp.float32)
    pool_i = jnp.dot(oh, out2, preferred_element_type=jnp.float32, precision=lax.Precision.HIGHEST)
    cnt_i = jnp.sum(oh, axis=1, keepdims=True)

    @pl.when(i == 0)
    def _():
        pool_acc[...] = pool_i
        cnt_acc[...] = cnt_i

    @pl.when(i > 0)
    def _():
        pool_acc[...] = pool_acc[...] + pool_i
        cnt_acc[...] = cnt_acc[...] + cnt_i

    @pl.when(i == NB - 1)
    def _():
        pooled = pool_acc[...] / jnp.maximum(cnt_acc[...], 1.0)
        col = lax.broadcasted_iota(jnp.int32, (G, H), 1)
        valid = col < C
        mx = jnp.max(jnp.where(valid, pooled, -1e30), axis=1, keepdims=True)
        ex = jnp.where(valid, jnp.exp(pooled - mx), 0.0)
        lse = mx + jnp.log(jnp.sum(ex, axis=1, keepdims=True))
        out_ref[...] = (pooled - lse)[:, :C]


def _tc3(z2a, z2b, ys, dinv, b2p, batch_r):
    return pl.pallas_call(
        _tc3_body,
        grid=(NB,),
        in_specs=[
            pl.BlockSpec((128, H), lambda i: (i, 0)),
            pl.BlockSpec((128, H), lambda i: (i, 0)),
            pl.BlockSpec((128, H), lambda i: (i, 0)),
            pl.BlockSpec((128, 1), lambda i: (i, 0)),
            pl.BlockSpec((1, H), lambda i: (0, 0)),
            pl.BlockSpec((1, 1, 128), lambda i: (i, 0, 0)),
        ],
        out_specs=pl.BlockSpec((G, C), lambda i: (0, 0)),
        out_shape=jax.ShapeDtypeStruct((G, C), jnp.float32),
        scratch_shapes=[
            pltpu.VMEM((G, H), jnp.float32),
            pltpu.VMEM((G, 1), jnp.float32),
        ],
        name="tc_pool_logsoftmax",
    )(z2a, z2b, ys, dinv, b2p, batch_r)


def kernel(x, edge_index, batch, W1, b1, W2, b2):
    f32 = jnp.float32
    src = edge_index[0].astype(jnp.int32)
    dst = edge_index[1].astype(jnp.int32)
    npad_e = E_PAD - E
    pad_rows = N + (jnp.arange(npad_e, dtype=jnp.int32) % (N_PAD - N))
    src_pad = jnp.concatenate([src, pad_rows])
    dst_pad = jnp.concatenate([dst, pad_rows])
    idx_slab = jnp.concatenate([src_pad, dst_pad]).reshape(2 * NW * CPW, CHUNK)
    x_pad = jnp.zeros((N_PAD, D), f32).at[:N].set(x.astype(f32))
    batch_pad = jnp.full((N_PAD,), G, jnp.int32).at[:N].set(batch.astype(jnp.int32))
    batch_r = batch_pad.reshape(NB, 1, 128)
    w2p = jnp.zeros((H, H), f32).at[:, :C].set(W2.astype(f32))
    b1r = b1.astype(f32).reshape(1, H)
    b2p = jnp.zeros((1, H), f32).at[0, :C].set(b2.astype(f32))

    deg_parts = _make_sc_deg()(idx_slab)
    d0 = deg_parts[:N_PAD].reshape(N_PAD, 1)
    d1 = deg_parts[N_PAD:].reshape(N_PAD, 1)

    dinv, h1s = _tc1(x_pad, W1.astype(f32), d0, d1)

    sc_agg = _make_sc_agg()
    z1 = sc_agg(h1s, idx_slab)
    z1a, z1b = z1[:N_PAD], z1[N_PAD:]

    ys = _tc2(z1a, z1b, h1s, dinv, b1r, w2p)

    z2 = sc_agg(ys, idx_slab)
    z2a, z2b = z2[:N_PAD], z2[N_PAD:]

    return _tc3(z2a, z2b, ys, dinv, b2p, batch_r)


# speedup vs baseline: 65.1445x; 1.9345x over previous
import jax
import jax.numpy as jnp
from jax import lax
from jax.experimental import pallas as pl
from jax.experimental.pallas import tpu as pltpu
from jax.experimental.pallas import tpu_sc as plsc

N = 10000
E = 320000
D = 128
H = 16
C = 10
G = 64

NC = 2
NS = 16
NW = NC * NS
CHUNK = 128
CPW = -(-E // (NW * CHUNK))
EPW = CPW * CHUNK
E_PAD = NW * EPW
N_PAD = 10240
RPW = N_PAD // NS
M = N_PAD // 8
MV = N // 8

NBUF = 8
LEAD = 4


def _sc_deg_body(idx_hbm, out_hbm, zbuf, ones_b, dslab, acc, *sems):
    c = lax.axis_index("c")
    s = lax.axis_index("s")
    wid = c * NS + s
    zero16 = jnp.zeros((16,), jnp.float32)
    one16 = jnp.ones((16,), jnp.float32)
    for j in range(CHUNK):
        zbuf[j, :] = zero16
        ones_b[j, :] = one16
    pltpu.sync_copy(idx_hbm.at[pl.ds((NW + wid) * CPW, CPW)], dslab)
    base_r = s * RPW
    for j in range(RPW // CHUNK):
        pltpu.sync_copy(zbuf, acc.at[pl.ds(base_r + j * CHUNK, CHUNK)])
    plsc.subcore_barrier()
    sd = {}
    for k in range(CPW):
        if k >= NBUF:
            sd[k - NBUF].wait()
        sd[k] = pltpu.async_copy(ones_b, acc.at[dslab.at[k]], sems[k % NBUF],
                                 add=True)
    for k in range(max(0, CPW - NBUF), CPW):
        sd[k].wait()
    plsc.subcore_barrier()
    pltpu.sync_copy(acc.at[pl.ds(base_r, RPW)],
                    out_hbm.at[pl.ds(c * N_PAD + base_r, RPW)])


def _make_sc_deg():
    return pl.kernel(
        _sc_deg_body,
        out_type=jax.ShapeDtypeStruct((NC * N_PAD, H), jnp.float32),
        mesh=plsc.VectorSubcoreMesh(core_axis_name="c", subcore_axis_name="s"),
        scratch_types=[
            pltpu.VMEM((CHUNK, H), jnp.float32),
            pltpu.VMEM((CHUNK, H), jnp.float32),
            pltpu.VMEM((CPW, CHUNK), jnp.int32),
            pltpu.VMEM_SHARED((N_PAD, H), jnp.float32),
        ] + [pltpu.SemaphoreType.DMA] * NBUF,
        compiler_params=pltpu.CompilerParams(use_tc_tiling_on_sc=False),
        name="sc_degree",
    )


def _sc_agg_body(feat_hbm, idx_hbm, out_hbm,
                 zbuf, sslab, dslab, rows, acc, *sems):
    gsem = sems[:NBUF]
    ssem = sems[NBUF:]
    c = lax.axis_index("c")
    s = lax.axis_index("s")
    wid = c * NS + s
    zero16 = jnp.zeros((16,), jnp.float32)
    for j in range(CHUNK):
        zbuf[j, :] = zero16
    pltpu.sync_copy(idx_hbm.at[pl.ds(wid * CPW, CPW)], sslab)
    pltpu.sync_copy(idx_hbm.at[pl.ds((NW + wid) * CPW, CPW)], dslab)
    base_r = s * RPW
    for j in range(RPW // CHUNK):
        pltpu.sync_copy(zbuf, acc.at[pl.ds(base_r + j * CHUNK, CHUNK)])
    plsc.subcore_barrier()

    gd, sd = {}, {}
    s_waited = set()

    def start_gather(k):
        b = k % NBUF
        gd[k] = pltpu.async_copy(feat_hbm.at[sslab.at[k]], rows.at[b], gsem[b])

    for j in range(min(LEAD, CPW)):
        start_gather(j)
    for k in range(CPW):
        b = k % NBUF
        nk = k + LEAD
        if nk < CPW:
            if nk >= NBUF:
                sd[nk - NBUF].wait()
                s_waited.add(nk - NBUF)
            start_gather(nk)
        gd[k].wait()
        sd[k] = pltpu.async_copy(rows.at[b], acc.at[dslab.at[k]], ssem[b],
                                 add=True)
    for k in range(CPW):
        if k not in s_waited:
            sd[k].wait()
    plsc.subcore_barrier()
    pltpu.sync_copy(acc.at[pl.ds(base_r, RPW)],
                    out_hbm.at[pl.ds(c * N_PAD + base_r, RPW)])


def _make_sc_agg():
    return pl.kernel(
        _sc_agg_body,
        out_type=jax.ShapeDtypeStruct((NC * N_PAD, H), jnp.float32),
        mesh=plsc.VectorSubcoreMesh(core_axis_name="c", subcore_axis_name="s"),
        scratch_types=[
            pltpu.VMEM((CHUNK, H), jnp.float32),
            pltpu.VMEM((CPW, CHUNK), jnp.int32),
            pltpu.VMEM((CPW, CHUNK), jnp.int32),
            pltpu.VMEM((NBUF, CHUNK, H), jnp.float32),
            pltpu.VMEM_SHARED((N_PAD, H), jnp.float32),
        ] + [pltpu.SemaphoreType.DMA] * (2 * NBUF),
        compiler_params=pltpu.CompilerParams(use_tc_tiling_on_sc=False),
        name="sc_edge_agg",
    )


def _tc1a_body(xb_ref, w1b_ref, h1_ref):
    h1_ref[...] = jnp.dot(xb_ref[...], w1b_ref[...],
                          preferred_element_type=jnp.float32,
                          precision=lax.Precision.HIGHEST)


def _tc1a(xb, w1b):
    return pl.pallas_call(
        _tc1a_body,
        out_shape=jax.ShapeDtypeStruct((M, 128), jnp.float32),
        name="tc_h1_matmul",
    )(xb, w1b)


def _tc1b_body(d0_ref, d1_ref, h1_ref, dinv_ref, h1s_ref):
    deg = d0_ref[...] + d1_ref[...] + 1.0
    m = lax.broadcasted_iota(jnp.int32, (M, 1), 0)
    dinv = jnp.where(m < MV, lax.rsqrt(deg), 0.0)
    dinv_ref[...] = dinv
    h1s_ref[...] = dinv * h1_ref[...]


def _tc1b(d0, d1, h1):
    return pl.pallas_call(
        _tc1b_body,
        out_shape=[
            jax.ShapeDtypeStruct((M, 128), jnp.float32),
            jax.ShapeDtypeStruct((M, 128), jnp.float32),
        ],
        name="tc_dinv_h1s",
    )(d0, d1, h1)


def _tc2_body(z0_ref, z1_ref, h1s_ref, dinv_ref, b1_ref, w2b_ref, ys_ref):
    dinv = dinv_ref[...]
    out1 = dinv * (z0_ref[...] + z1_ref[...] + h1s_ref[...]) + b1_ref[...]
    h = jnp.maximum(out1, 0.0)
    ys_ref[...] = dinv * jnp.dot(h, w2b_ref[...],
                                 preferred_element_type=jnp.float32,
                                 precision=lax.Precision.HIGHEST)


def _tc2(z0, z1, h1s, dinv, b1f, w2b):
    return pl.pallas_call(
        _tc2_body,
        out_shape=jax.ShapeDtypeStruct((M, 128), jnp.float32),
        name="tc_layer1_finish",
    )(z0, z1, h1s, dinv, b1f, w2b)


def _tc3_body(z0_ref, z1_ref, ys_ref, dinv_ref, b2_ref, batch8_ref, out_ref):
    dinv = dinv_ref[...]
    out2 = dinv * (z0_ref[...] + z1_ref[...] + ys_ref[...]) + b2_ref[...]
    b8 = batch8_ref[...]
    gid = lax.broadcasted_iota(jnp.int32, (G, 1), 0)
    pool = jnp.zeros((G, H), jnp.float32)
    cnt = jnp.zeros((G, 1), jnp.float32)
    for a in range(8):
        oh = (b8[a:a + 1, :] == gid).astype(jnp.float32)
        pool = pool + jnp.dot(oh, out2[:, 16 * a:16 * a + 16],
                              preferred_element_type=jnp.float32,
                              precision=lax.Precision.HIGHEST)
        cnt = cnt + jnp.sum(oh, axis=1, keepdims=True)
    pooled = pool / jnp.maximum(cnt, 1.0)
    col = lax.broadcasted_iota(jnp.int32, (G, H), 1)
    valid = col < C
    mx = jnp.max(jnp.where(valid, pooled, -1e30), axis=1, keepdims=True)
    ex = jnp.where(valid, jnp.exp(pooled - mx), 0.0)
    lse = mx + jnp.log(jnp.sum(ex, axis=1, keepdims=True))
    out_ref[...] = (pooled - lse)[:, :C]


def _tc3(z0, z1, ys, dinv, b2f, batch8):
    return pl.pallas_call(
        _tc3_body,
        out_shape=jax.ShapeDtypeStruct((G, C), jnp.float32),
        name="tc_pool_logsoftmax",
    )(z0, z1, ys, dinv, b2f, batch8)


def kernel(x, edge_index, batch, W1, b1, W2, b2):
    f32 = jnp.float32
    src = edge_index[0].astype(jnp.int32)
    dst = edge_index[1].astype(jnp.int32)
    npad_e = E_PAD - E
    pad_rows = N + (jnp.arange(npad_e, dtype=jnp.int32) % (N_PAD - N))
    src_pad = jnp.concatenate([src, pad_rows])
    dst_pad = jnp.concatenate([dst, pad_rows])
    idx_slab = jnp.concatenate([src_pad, dst_pad]).reshape(2 * NW * CPW, CHUNK)

    xb = jnp.zeros((N_PAD, D), f32).at[:N].set(x.astype(f32))
    xb = xb.reshape(M, 8 * D)
    w1b = jnp.kron(jnp.eye(8, dtype=f32), W1.astype(f32))
    w2p = jnp.zeros((H, H), f32).at[:, :C].set(W2.astype(f32))
    w2b = jnp.kron(jnp.eye(8, dtype=f32), w2p)
    b1f = jnp.tile(b1.astype(f32), 8).reshape(1, 128)
    b2p = jnp.zeros((H,), f32).at[:C].set(b2.astype(f32))
    b2f = jnp.tile(b2p, 8).reshape(1, 128)
    batch_pad = jnp.full((N_PAD,), G, jnp.int32).at[:N].set(batch.astype(jnp.int32))
    batch8 = batch_pad.reshape(M, 8).T

    deg_parts = _make_sc_deg()(idx_slab).reshape(NC, M, 128)

    h1 = _tc1a(xb, w1b)
    dinv, h1s = _tc1b(deg_parts[0], deg_parts[1], h1)

    sc_agg = _make_sc_agg()
    z1 = sc_agg(h1s.reshape(N_PAD, H), idx_slab).reshape(NC, M, 128)

    ys = _tc2(z1[0], z1[1], h1s, dinv, b1f, w2b)

    z2 = sc_agg(ys.reshape(N_PAD, H), idx_slab).reshape(NC, M, 128)

    return _tc3(z2[0], z2[1], ys, dinv, b2f, batch8)


# speedup vs baseline: 100.0370x; 1.5356x over previous
import jax
import jax.numpy as jnp
from jax import lax
from jax.experimental import pallas as pl
from jax.experimental.pallas import tpu as pltpu
from jax.experimental.pallas import tpu_sc as plsc

N = 10000
E = 320000
D = 128
H = 16
C = 10
G = 64

NC = 2
NS = 16
NW = NC * NS
CHUNK = 128
CPW = -(-E // (NW * CHUNK))
EPW = CPW * CHUNK
E_PAD = NW * EPW
N_PAD = 10240
RPW = N_PAD // NS
M = N_PAD // 8
MV = N // 8

NBUF = 8
LEAD = 4


def _sc_deg_body(idx_hbm, out_hbm, zbuf, ones_b, dslab, acc, *sems):
    c = lax.axis_index("c")
    s = lax.axis_index("s")
    wid = c * NS + s
    zero16 = jnp.zeros((16,), jnp.float32)
    one16 = jnp.ones((16,), jnp.float32)
    for j in range(CHUNK):
        zbuf[j, :] = zero16
        ones_b[j, :] = one16
    pltpu.sync_copy(idx_hbm.at[pl.ds((NW + wid) * CPW, CPW)], dslab)
    base_r = s * RPW
    for j in range(RPW // CHUNK):
        pltpu.sync_copy(zbuf, acc.at[pl.ds(base_r + j * CHUNK, CHUNK)])
    plsc.subcore_barrier()
    sd = {}
    for k in range(CPW):
        if k >= NBUF:
            sd[k - NBUF].wait()
        sd[k] = pltpu.async_copy(ones_b, acc.at[dslab.at[k]], sems[k % NBUF],
                                 add=True)
    for k in range(max(0, CPW - NBUF), CPW):
        sd[k].wait()
    plsc.subcore_barrier()
    pltpu.sync_copy(acc.at[pl.ds(base_r, RPW)],
                    out_hbm.at[pl.ds(c * N_PAD + base_r, RPW)])


def _make_sc_deg():
    return pl.kernel(
        _sc_deg_body,
        out_type=jax.ShapeDtypeStruct((NC * N_PAD, H), jnp.float32),
        mesh=plsc.VectorSubcoreMesh(core_axis_name="c", subcore_axis_name="s"),
        scratch_types=[
            pltpu.VMEM((CHUNK, H), jnp.float32),
            pltpu.VMEM((CHUNK, H), jnp.float32),
            pltpu.VMEM((CPW, CHUNK), jnp.int32),
            pltpu.VMEM_SHARED((N_PAD, H), jnp.float32),
        ] + [pltpu.SemaphoreType.DMA] * NBUF,
        compiler_params=pltpu.CompilerParams(use_tc_tiling_on_sc=False),
        name="sc_degree",
    )


def _sc_agg_body(feat_hbm, idx_hbm, out_hbm,
                 zbuf, sslab, dslab, rows, acc, *sems):
    gsem = sems[:NBUF]
    ssem = sems[NBUF:]
    c = lax.axis_index("c")
    s = lax.axis_index("s")
    wid = c * NS + s
    zero16 = jnp.zeros((16,), jnp.float32)
    for j in range(CHUNK):
        zbuf[j, :] = zero16
    pltpu.sync_copy(idx_hbm.at[pl.ds(wid * CPW, CPW)], sslab)
    pltpu.sync_copy(idx_hbm.at[pl.ds((NW + wid) * CPW, CPW)], dslab)
    base_r = s * RPW
    for j in range(RPW // CHUNK):
        pltpu.sync_copy(zbuf, acc.at[pl.ds(base_r + j * CHUNK, CHUNK)])
    plsc.subcore_barrier()

    gd, sd = {}, {}
    s_waited = set()

    def start_gather(k):
        b = k % NBUF
        gd[k] = pltpu.async_copy(feat_hbm.at[sslab.at[k]], rows.at[b], gsem[b])

    for j in range(min(LEAD, CPW)):
        start_gather(j)
    for k in range(CPW):
        b = k % NBUF
        nk = k + LEAD
        if nk < CPW:
            if nk >= NBUF:
                sd[nk - NBUF].wait()
                s_waited.add(nk - NBUF)
            start_gather(nk)
        gd[k].wait()
        sd[k] = pltpu.async_copy(rows.at[b], acc.at[dslab.at[k]], ssem[b],
                                 add=True)
    for k in range(CPW):
        if k not in s_waited:
            sd[k].wait()
    plsc.subcore_barrier()
    pltpu.sync_copy(acc.at[pl.ds(base_r, RPW)],
                    out_hbm.at[pl.ds(c * N_PAD + base_r, RPW)])


def _make_sc_agg():
    return pl.kernel(
        _sc_agg_body,
        out_type=jax.ShapeDtypeStruct((NC * N_PAD, H), jnp.float32),
        mesh=plsc.VectorSubcoreMesh(core_axis_name="c", subcore_axis_name="s"),
        scratch_types=[
            pltpu.VMEM((CHUNK, H), jnp.float32),
            pltpu.VMEM((CPW, CHUNK), jnp.int32),
            pltpu.VMEM((CPW, CHUNK), jnp.int32),
            pltpu.VMEM((NBUF, CHUNK, H), jnp.float32),
            pltpu.VMEM_SHARED((N_PAD, H), jnp.float32),
        ] + [pltpu.SemaphoreType.DMA] * (2 * NBUF),
        compiler_params=pltpu.CompilerParams(use_tc_tiling_on_sc=False),
        name="sc_edge_agg",
    )


def _tc1a_body(xb_ref, w1b_ref, h1_ref):
    h1_ref[...] = jnp.dot(xb_ref[...], w1b_ref[...],
                          preferred_element_type=jnp.float32,
                          precision=lax.Precision.HIGHEST)


def _tc1a(xb, w1b):
    return pl.pallas_call(
        _tc1a_body,
        out_shape=jax.ShapeDtypeStruct((MV, 128), jnp.float32),
        name="tc_h1_matmul",
    )(xb, w1b)


def _tc1b_body(deg_ref, h1_ref, dinv_ref, h1s_ref):
    deg = deg_ref[:M] + deg_ref[M:] + 1.0
    m = lax.broadcasted_iota(jnp.int32, (M, 1), 0)
    dinv = jnp.where(m < MV, lax.rsqrt(deg), 0.0)
    dinv_ref[...] = dinv
    h1s_ref[...] = dinv * h1_ref[...]


def _tc1b(deg, h1):
    return pl.pallas_call(
        _tc1b_body,
        out_shape=[
            jax.ShapeDtypeStruct((M, 128), jnp.float32),
            jax.ShapeDtypeStruct((M, 128), jnp.float32),
        ],
        name="tc_dinv_h1s",
    )(deg, h1)


def _tc2_body(z_ref, h1s_ref, dinv_ref, b1_ref, w2b_ref, ys_ref):
    dinv = dinv_ref[...]
    out1 = dinv * (z_ref[:M] + z_ref[M:] + h1s_ref[...]) + b1_ref[...]
    h = jnp.maximum(out1, 0.0)
    ys_ref[...] = dinv * jnp.dot(h, w2b_ref[...],
                                 preferred_element_type=jnp.float32,
                                 precision=lax.Precision.HIGHEST)


def _tc2(z, h1s, dinv, b1f, w2b):
    return pl.pallas_call(
        _tc2_body,
        out_shape=jax.ShapeDtypeStruct((M, 128), jnp.float32),
        name="tc_layer1_finish",
    )(z, h1s, dinv, b1f, w2b)


def _tc3_body(z_ref, ys_ref, dinv_ref, b2_ref, batch8_ref, out_ref):
    dinv = dinv_ref[...]
    out2 = dinv * (z_ref[:M] + z_ref[M:] + ys_ref[...]) + b2_ref[...]
    b8 = batch8_ref[...]
    gid = lax.broadcasted_iota(jnp.int32, (G, 1), 0)
    pool = jnp.zeros((G, H), jnp.float32)
    cnt = jnp.zeros((G, 1), jnp.float32)
    for a in range(8):
        oh = (b8[a:a + 1, :] == gid).astype(jnp.float32)
        pool = pool + jnp.dot(oh, out2[:, 16 * a:16 * a + 16],
                              preferred_element_type=jnp.float32,
                              precision=lax.Precision.HIGHEST)
        cnt = cnt + jnp.sum(oh, axis=1, keepdims=True)
    pooled = pool / jnp.maximum(cnt, 1.0)
    col = lax.broadcasted_iota(jnp.int32, (G, H), 1)
    valid = col < C
    mx = jnp.max(jnp.where(valid, pooled, -1e30), axis=1, keepdims=True)
    ex = jnp.where(valid, jnp.exp(pooled - mx), 0.0)
    lse = mx + jnp.log(jnp.sum(ex, axis=1, keepdims=True))
    out_ref[...] = (pooled - lse)[:, :C]


def _tc3(z, ys, dinv, b2f, batch8):
    return pl.pallas_call(
        _tc3_body,
        out_shape=jax.ShapeDtypeStruct((G, C), jnp.float32),
        name="tc_pool_logsoftmax",
    )(z, ys, dinv, b2f, batch8)


def kernel(x, edge_index, batch, W1, b1, W2, b2):
    f32 = jnp.float32
    npad_r = NW * CPW - E // CHUNK
    pad_blk = N + (jnp.arange(npad_r * CHUNK, dtype=jnp.int32)
                   % (N_PAD - N)).reshape(1, npad_r, CHUNK)
    pad_blk = jnp.broadcast_to(pad_blk, (2, npad_r, CHUNK))
    ei = edge_index.astype(jnp.int32).reshape(2, E // CHUNK, CHUNK)
    idx_slab = jnp.concatenate([ei, pad_blk], axis=1).reshape(2 * NW * CPW,
                                                              CHUNK)

    xb = x.astype(f32).reshape(MV, 8 * D)
    w1b = jnp.kron(jnp.eye(8, dtype=f32), W1.astype(f32))
    w2p = jnp.zeros((H, H), f32).at[:, :C].set(W2.astype(f32))
    w2b = jnp.kron(jnp.eye(8, dtype=f32), w2p)
    b1f = jnp.tile(b1.astype(f32), 8).reshape(1, 128)
    b2p = jnp.zeros((H,), f32).at[:C].set(b2.astype(f32))
    b2f = jnp.tile(b2p, 8).reshape(1, 128)
    batch_pad = jnp.full((N_PAD,), G, jnp.int32).at[:N].set(batch.astype(jnp.int32))
    batch8 = batch_pad.reshape(M, 8).T

    deg = _make_sc_deg()(idx_slab).reshape(-1).reshape(NC * M, 128)

    h1 = _tc1a(xb, w1b)
    h1p = jnp.zeros((M, 128), f32).at[:MV].set(h1)
    dinv, h1s = _tc1b(deg, h1p)

    sc_agg = _make_sc_agg()
    feat1 = h1s.reshape(-1).reshape(N_PAD, H)
    z1 = sc_agg(feat1, idx_slab).reshape(-1).reshape(NC * M, 128)

    ys = _tc2(z1, h1s, dinv, b1f, w2b)

    feat2 = ys.reshape(-1).reshape(N_PAD, H)
    z2 = sc_agg(feat2, idx_slab).reshape(-1).reshape(NC * M, 128)

    return _tc3(z2, ys, dinv, b2f, batch8)


# speedup vs baseline: 103.8579x; 1.0382x over previous
import jax
import jax.numpy as jnp
from jax import lax
from jax.experimental import pallas as pl
from jax.experimental.pallas import tpu as pltpu
from jax.experimental.pallas import tpu_sc as plsc

N = 10000
E = 320000
D = 128
H = 16
C = 10
G = 64

NC = 2
NS = 16
NW = NC * NS
CHUNK = 128
CPW = 80
EPW = CPW * CHUNK
E_PAD = NW * EPW
N_PAD = 10240
RPW = N_PAD // NS
M = N_PAD // 8
MV = N // 8

NBUF = 8
GRP = 8
NGRP = CPW // GRP
ANBUF = 4
ALEAD = 2


def _sc_deg_body(idx_hbm, out_hbm, zbuf, ones_b, dslab, acc, *sems):
    c = lax.axis_index("c")
    s = lax.axis_index("s")
    wid = c * NS + s
    zero16 = jnp.zeros((16,), jnp.float32)
    one16 = jnp.ones((16,), jnp.float32)
    for j in range(CHUNK):
        zbuf[j, :] = zero16
    for j in range(GRP * CHUNK):
        ones_b[j, :] = one16
    pltpu.sync_copy(idx_hbm.at[pl.ds((NW + wid) * NGRP, NGRP)], dslab)
    base_r = s * RPW
    for j in range(RPW // CHUNK):
        pltpu.sync_copy(zbuf, acc.at[pl.ds(base_r + j * CHUNK, CHUNK)])
    plsc.subcore_barrier()
    sd = {}
    for g in range(NGRP):
        if g >= NBUF:
            sd[g - NBUF].wait()
        sd[g] = pltpu.async_copy(ones_b, acc.at[dslab.at[g]],
                                 sems[g % NBUF], add=True)
    for g in range(max(0, NGRP - NBUF), NGRP):
        sd[g].wait()
    plsc.subcore_barrier()
    pltpu.sync_copy(acc.at[pl.ds(base_r, RPW)],
                    out_hbm.at[pl.ds(c * N_PAD + base_r, RPW)])


def _make_sc_deg():
    return pl.kernel(
        _sc_deg_body,
        out_type=jax.ShapeDtypeStruct((NC * N_PAD, H), jnp.float32),
        mesh=plsc.VectorSubcoreMesh(core_axis_name="c", subcore_axis_name="s"),
        scratch_types=[
            pltpu.VMEM((CHUNK, H), jnp.float32),
            pltpu.VMEM((GRP * CHUNK, H), jnp.float32),
            pltpu.VMEM((NGRP, GRP * CHUNK), jnp.int32),
            pltpu.VMEM_SHARED((N_PAD, H), jnp.float32),
        ] + [pltpu.SemaphoreType.DMA] * NBUF,
        compiler_params=pltpu.CompilerParams(use_tc_tiling_on_sc=False),
        name="sc_degree",
    )


def _sc_agg_body(feat_hbm, idx_hbm, out_hbm,
                 zbuf, sslab, dslab, rows, acc, *sems):
    gsem = sems[:ANBUF]
    ssem = sems[ANBUF:]
    c = lax.axis_index("c")
    s = lax.axis_index("s")
    wid = c * NS + s
    zero16 = jnp.zeros((16,), jnp.float32)
    for j in range(CHUNK):
        zbuf[j, :] = zero16
    pltpu.sync_copy(idx_hbm.at[pl.ds(wid * NGRP, NGRP)], sslab)
    pltpu.sync_copy(idx_hbm.at[pl.ds((NW + wid) * NGRP, NGRP)], dslab)
    base_r = s * RPW
    for j in range(RPW // CHUNK):
        pltpu.sync_copy(zbuf, acc.at[pl.ds(base_r + j * CHUNK, CHUNK)])
    plsc.subcore_barrier()

    gd, sd = {}, {}
    s_waited = set()

    def start_gather(g):
        b = g % ANBUF
        gd[g] = pltpu.async_copy(feat_hbm.at[sslab.at[g]], rows.at[b], gsem[b])

    for j in range(min(ALEAD, NGRP)):
        start_gather(j)
    for g in range(NGRP):
        b = g % ANBUF
        ng = g + ALEAD
        if ng < NGRP:
            if ng >= ANBUF:
                sd[ng - ANBUF].wait()
                s_waited.add(ng - ANBUF)
            start_gather(ng)
        gd[g].wait()
        sd[g] = pltpu.async_copy(rows.at[b], acc.at[dslab.at[g]], ssem[b],
                                 add=True)
    for g in range(NGRP):
        if g not in s_waited:
            sd[g].wait()
    plsc.subcore_barrier()
    pltpu.sync_copy(acc.at[pl.ds(base_r, RPW)],
                    out_hbm.at[pl.ds(c * N_PAD + base_r, RPW)])


def _make_sc_agg():
    return pl.kernel(
        _sc_agg_body,
        out_type=jax.ShapeDtypeStruct((NC * N_PAD, H), jnp.float32),
        mesh=plsc.VectorSubcoreMesh(core_axis_name="c", subcore_axis_name="s"),
        scratch_types=[
            pltpu.VMEM((CHUNK, H), jnp.float32),
            pltpu.VMEM((NGRP, GRP * CHUNK), jnp.int32),
            pltpu.VMEM((NGRP, GRP * CHUNK), jnp.int32),
            pltpu.VMEM((ANBUF, GRP * CHUNK, H), jnp.float32),
            pltpu.VMEM_SHARED((N_PAD, H), jnp.float32),
        ] + [pltpu.SemaphoreType.DMA] * (2 * ANBUF),
        compiler_params=pltpu.CompilerParams(use_tc_tiling_on_sc=False),
        name="sc_edge_agg",
    )


def _tc1a_body(xb_ref, w1b_ref, h1_ref):
    h1_ref[...] = jnp.dot(xb_ref[...], w1b_ref[...],
                          preferred_element_type=jnp.float32,
                          precision=lax.Precision.HIGHEST)


def _tc1a(xb, w1b):
    return pl.pallas_call(
        _tc1a_body,
        out_shape=jax.ShapeDtypeStruct((M, 128), jnp.float32),
        name="tc_h1_matmul",
    )(xb, w1b)


def _tc1b_body(deg_ref, h1_ref, dinv_ref, h1s_ref):
    deg = deg_ref[:M] + deg_ref[M:] + 1.0
    m = lax.broadcasted_iota(jnp.int32, (M, 1), 0)
    dinv = jnp.where(m < MV, lax.rsqrt(deg), 0.0)
    dinv_ref[...] = dinv
    h1s_ref[...] = dinv * h1_ref[...]


def _tc1b(deg, h1):
    return pl.pallas_call(
        _tc1b_body,
        out_shape=[
            jax.ShapeDtypeStruct((M, 128), jnp.float32),
            jax.ShapeDtypeStruct((M, 128), jnp.float32),
        ],
        name="tc_dinv_h1s",
    )(deg, h1)


def _tc2_body(z_ref, h1s_ref, dinv_ref, b1_ref, w2b_ref, ys_ref):
    dinv = dinv_ref[...]
    out1 = dinv * (z_ref[:M] + z_ref[M:] + h1s_ref[...]) + b1_ref[...]
    h = jnp.maximum(out1, 0.0)
    ys_ref[...] = dinv * jnp.dot(h, w2b_ref[...],
                                 preferred_element_type=jnp.float32,
                                 precision=lax.Precision.HIGHEST)


def _tc2(z, h1s, dinv, b1f, w2b):
    return pl.pallas_call(
        _tc2_body,
        out_shape=jax.ShapeDtypeStruct((M, 128), jnp.float32),
        name="tc_layer1_finish",
    )(z, h1s, dinv, b1f, w2b)


def _tc3_body(z_ref, ys_ref, dinv_ref, b2_ref, batch8_ref, out_ref):
    dinv = dinv_ref[...]
    out2 = dinv * (z_ref[:M] + z_ref[M:] + ys_ref[...]) + b2_ref[...]
    b8 = batch8_ref[...]
    gid = lax.broadcasted_iota(jnp.int32, (G, 1), 0)
    pool = jnp.zeros((G, H), jnp.float32)
    cnt = jnp.zeros((G, 1), jnp.float32)
    for a in range(8):
        oh = (b8[a:a + 1, :] == gid).astype(jnp.float32)
        pool = pool + jnp.dot(oh, out2[:, 16 * a:16 * a + 16],
                              preferred_element_type=jnp.float32,
                              precision=lax.Precision.HIGHEST)
        cnt = cnt + jnp.sum(oh, axis=1, keepdims=True)
    pooled = pool / jnp.maximum(cnt, 1.0)
    col = lax.broadcasted_iota(jnp.int32, (G, H), 1)
    valid = col < C
    mx = jnp.max(jnp.where(valid, pooled, -1e30), axis=1, keepdims=True)
    ex = jnp.where(valid, jnp.exp(pooled - mx), 0.0)
    lse = mx + jnp.log(jnp.sum(ex, axis=1, keepdims=True))
    out_ref[...] = (pooled - lse)[:, :C]


def _tc3(z, ys, dinv, b2f, batch8):
    return pl.pallas_call(
        _tc3_body,
        out_shape=jax.ShapeDtypeStruct((G, C), jnp.float32),
        name="tc_pool_logsoftmax",
    )(z, ys, dinv, b2f, batch8)


def kernel(x, edge_index, batch, W1, b1, W2, b2):
    f32 = jnp.float32
    npad_r = NW * CPW - E // CHUNK
    pad_blk = N + (jnp.arange(npad_r * CHUNK, dtype=jnp.int32)
                   % (N_PAD - N)).reshape(1, npad_r, CHUNK)
    pad_blk = jnp.broadcast_to(pad_blk, (2, npad_r, CHUNK))
    ei = edge_index.astype(jnp.int32).reshape(2, E // CHUNK, CHUNK)
    idx_slab = jnp.concatenate([ei, pad_blk], axis=1).reshape(-1)
    idx_slab = idx_slab.reshape(2 * NW * NGRP, GRP * CHUNK)

    xflat = x.astype(f32).reshape(-1)
    xb = jnp.concatenate([xflat, jnp.zeros(((M - MV) * 8 * D,), f32)])
    xb = xb.reshape(M, 8 * D)
    w1b = jnp.kron(jnp.eye(8, dtype=f32), W1.astype(f32))
    w2p = jnp.zeros((H, H), f32).at[:, :C].set(W2.astype(f32))
    w2b = jnp.kron(jnp.eye(8, dtype=f32), w2p)
    b1f = jnp.tile(b1.astype(f32), 8).reshape(1, 128)
    b2p = jnp.zeros((H,), f32).at[:C].set(b2.astype(f32))
    b2f = jnp.tile(b2p, 8).reshape(1, 128)
    batch_pad = jnp.full((N_PAD,), G, jnp.int32).at[:N].set(batch.astype(jnp.int32))
    batch8 = batch_pad.reshape(M, 8).T

    deg = _make_sc_deg()(idx_slab).reshape(-1).reshape(NC * M, 128)

    h1 = _tc1a(xb, w1b)
    dinv, h1s = _tc1b(deg, h1)

    sc_agg = _make_sc_agg()
    feat1 = h1s.reshape(-1).reshape(N_PAD, H)
    z1 = sc_agg(feat1, idx_slab).reshape(-1).reshape(NC * M, 128)

    ys = _tc2(z1, h1s, dinv, b1f, w2b)

    feat2 = ys.reshape(-1).reshape(N_PAD, H)
    z2 = sc_agg(feat2, idx_slab).reshape(-1).reshape(NC * M, 128)

    return _tc3(z2, ys, dinv, b2f, batch8)
